# Initial kernel scaffold; baseline (speedup 1.0000x reference)
import jax
import jax.numpy as jnp
from jax.experimental import pallas as pl


def kernel(x, loc, h, w):
    raise NotImplementedError("write your pallas kernel here")


# speedup vs baseline: 1.9939x; 1.9939x over previous
import functools
import math

import jax
import jax.numpy as jnp
from jax import lax
from jax.experimental import pallas as pl
from jax.experimental.pallas import tpu as pltpu
from jax.experimental.pallas import tpu_sc as plsc

_SCALE_FACTOR = 2
_SIGMA = 2.0

_NC = 2
_NS = 16
_NW = _NC * _NS
_L = 16


def _prep_body(x_ref, loc_ref, scale_ref, idx_ref, xt_ref):
    n, c = x_ref.shape[1], x_ref.shape[2]
    xb = x_ref[0]
    eye = (lax.broadcasted_iota(jnp.int32, (c, c), 0)
           == lax.broadcasted_iota(jnp.int32, (c, c), 1)).astype(jnp.float32)
    xt = lax.dot_general(eye, xb, (((1,), (1,)), ((), ())),
                         precision=lax.Precision.HIGHEST,
                         preferred_element_type=jnp.float32)
    xt_ref[0] = jnp.concatenate(
        [xt, jnp.ones((1, n), jnp.float32)], axis=0)

    eye2 = (lax.broadcasted_iota(jnp.int32, (2, 2), 0)
            == lax.broadcasted_iota(jnp.int32, (2, 2), 1)).astype(jnp.float32)
    loct = lax.dot_general(eye2, loc_ref[0], (((1,), (1,)), ((), ())),
                           precision=lax.Precision.HIGHEST,
                           preferred_element_type=jnp.float32)
    l = jnp.clip(loct, -1.0, 1.0)
    s = scale_ref[...]
    pix = 0.5 * (l + 1.0) * s - 0.5
    ri = jnp.round(pix).astype(jnp.int32)
    ri = jnp.clip(ri, 0, s.astype(jnp.int32) - 1)
    wv = jnp.concatenate(
        [jnp.ones((1, 1), jnp.int32), s.astype(jnp.int32)[0:1, :]], axis=0)
    idx_ref[0, 0] = jnp.sum(ri * wv, axis=0)


def _prep(x, loc, scale):
    b, n, c = x.shape
    return pl.pallas_call(
        _prep_body,
        grid=(b,),
        in_specs=[
            pl.BlockSpec((1, n, c), lambda i: (i, 0, 0)),
            pl.BlockSpec((1, n, 2), lambda i: (i, 0, 0)),
            pl.BlockSpec((2, 1), lambda i: (0, 0)),
        ],
        out_specs=[
            pl.BlockSpec((1, 1, n), lambda i: (i, 0, 0)),
            pl.BlockSpec((1, c + 1, n), lambda i: (i, 0, 0)),
        ],
        out_shape=[
            jax.ShapeDtypeStruct((b, 1, n), jnp.int32),
            jax.ShapeDtypeStruct((b, c + 1, n), jnp.float32),
        ],
    )(x, loc, scale)


def _scatter_sc(xt, idx3, hw):
    b, c1, n = xt.shape
    pairs = b * c1
    iters = (pairs + _NW - 1) // _NW
    groups = n // _L
    mesh = plsc.VectorSubcoreMesh(core_axis_name="c", subcore_axis_name="s")

    @functools.partial(
        pl.kernel,
        mesh=mesh,
        compiler_params=pltpu.CompilerParams(needs_layout_passes=False),
        out_type=jax.ShapeDtypeStruct((b, c1, hw), jnp.float32),
        scratch_types=[
            pltpu.VMEM((n,), jnp.int32),
            pltpu.VMEM((n,), jnp.float32),
            pltpu.VMEM((hw,), jnp.float32),
            pltpu.SemaphoreType.DMA,
            pltpu.SemaphoreType.DMA,
        ],
    )
    def scatter_kernel(xt_hbm, idx_hbm, out_hbm, idx_v, val_v, acc_v,
                       sem_i, sem_v):
        wid = lax.axis_index("s") * _NC + lax.axis_index("c")

        def pair_body(k, _):
            p = k * _NW + wid

            @pl.when(p < pairs)
            def _():
                bb = lax.div(p, c1)
                cc = lax.rem(p, c1)
                cp_i = pltpu.async_copy(idx_hbm.at[bb, 0], idx_v, sem_i)
                cp_v = pltpu.async_copy(xt_hbm.at[bb, cc], val_v, sem_v)

                zero = jnp.zeros((_L,), jnp.float32)

                def zero_body(z, _):
                    base = z * (8 * _L)
                    for u in range(8):
                        acc_v[pl.ds(base + u * _L, _L)] = zero
                    return ()

                lax.fori_loop(0, hw // (8 * _L), zero_body, ())
                cp_i.wait()
                cp_v.wait()

                def scat_body(g, _):
                    base = g * (4 * _L)
                    for u in range(4):
                        o = base + u * _L
                        iv = idx_v[pl.ds(o, _L)]
                        vv = val_v[pl.ds(o, _L)]
                        plsc.addupdate_scatter(acc_v, [iv], vv)
                    return ()

                lax.fori_loop(0, groups // 4, scat_body, ())
                pltpu.sync_copy(acc_v, out_hbm.at[bb, cc])

            return ()

        lax.fori_loop(0, iters, pair_body, ())

    return scatter_kernel(xt, idx3)


def _sep3(t, w0, w1):
    p = jnp.pad(t, ((0, 0), (1, 1), (0, 0)))
    t = w0 * p[:, :-2, :] + w1 * p[:, 1:-1, :] + w0 * p[:, 2:, :]
    p = jnp.pad(t, ((0, 0), (0, 0), (1, 1)))
    return w0 * p[:, :, :-2] + w1 * p[:, :, 1:-1] + w0 * p[:, :, 2:]


def _finish_body(f_ref, m_ref, o_ref, *, w0, w1):
    cnt = m_ref[0, 0]
    mask = (cnt > 0).astype(jnp.float32)
    feat = f_ref[0] / (cnt + 1e-6)[None] * mask[None]
    cf = _sep3(feat, w0, w1)
    cm = _sep3(mask[None], w0, w1)
    interp = cf / (cm + 1e-6) * (cm > 0).astype(jnp.float32)
    o_ref[0] = feat + (1.0 - mask)[None] * interp


def _finish(accum4):
    b, c1, hs, ws = accum4.shape
    c = c1 - 1
    cch = 12
    a = math.exp(-1.0 / (2.0 * _SIGMA * _SIGMA))
    w1 = 1.0 / (2.0 * a + 1.0)
    w0 = a * w1
    return pl.pallas_call(
        functools.partial(_finish_body, w0=w0, w1=w1),
        grid=(b, c // cch),
        in_specs=[
            pl.BlockSpec((1, cch, hs, ws), lambda i, j: (i, j, 0, 0)),
            pl.BlockSpec((1, 1, hs, ws), lambda i, j: (i, c, 0, 0)),
        ],
        out_specs=pl.BlockSpec((1, cch, hs, ws), lambda i, j: (i, j, 0, 0)),
        out_shape=jax.ShapeDtypeStruct((b, c, hs, ws), jnp.float32),
    )(accum4, accum4)


def kernel(x, loc, h, w):
    b, n, c = x.shape
    hs = math.isqrt(n) * _SCALE_FACTOR
    scale = jnp.stack([jnp.asarray(w, jnp.float32) * _SCALE_FACTOR,
                       jnp.asarray(h, jnp.float32) * _SCALE_FACTOR]
                      ).reshape(2, 1)
    idx3, xt = _prep(x, loc, scale)
    accum = _scatter_sc(xt, idx3, hs * hs)
    return _finish(accum.reshape(b, c + 1, hs, hs))


# speedup vs baseline: 2.6249x; 1.3165x over previous
import functools
import math

import jax
import jax.numpy as jnp
from jax import lax
from jax.experimental import pallas as pl
from jax.experimental.pallas import tpu as pltpu
from jax.experimental.pallas import tpu_sc as plsc

_SCALE_FACTOR = 2
_SIGMA = 2.0

_NC = 2
_NS = 16
_NW = _NC * _NS
_L = 16


def _prep_body(x_ref, loct_ref, scale_ref, idx_ref, xt_ref):
    n, c = x_ref.shape[1], x_ref.shape[2]
    xb = x_ref[0]
    eye = (lax.broadcasted_iota(jnp.int32, (c, c), 0)
           == lax.broadcasted_iota(jnp.int32, (c, c), 1)).astype(jnp.float32)
    xt = lax.dot_general(eye, xb, (((1,), (1,)), ((), ())),
                         precision=lax.Precision.HIGHEST,
                         preferred_element_type=jnp.float32)
    xt_ref[0] = jnp.concatenate(
        [xt, jnp.ones((1, n), jnp.float32)], axis=0)

    l = jnp.clip(loct_ref[0], -1.0, 1.0)
    s = scale_ref[...]
    pix = 0.5 * (l + 1.0) * s - 0.5
    ri = jnp.round(pix).astype(jnp.int32)
    ri = jnp.clip(ri, 0, s.astype(jnp.int32) - 1)
    wv = jnp.concatenate(
        [jnp.ones((1, 1), jnp.int32), s.astype(jnp.int32)[0:1, :]], axis=0)
    idx_ref[0, 0] = jnp.sum(ri * wv, axis=0)


def _prep(x, loct, scale):
    b, n, c = x.shape
    return pl.pallas_call(
        _prep_body,
        grid=(b,),
        in_specs=[
            pl.BlockSpec((1, n, c), lambda i: (i, 0, 0)),
            pl.BlockSpec((1, 2, n), lambda i: (i, 0, 0)),
            pl.BlockSpec((2, 1), lambda i: (0, 0)),
        ],
        out_specs=[
            pl.BlockSpec((1, 1, n), lambda i: (i, 0, 0)),
            pl.BlockSpec((1, c + 1, n), lambda i: (i, 0, 0)),
        ],
        out_shape=[
            jax.ShapeDtypeStruct((b, 1, n), jnp.int32),
            jax.ShapeDtypeStruct((b, c + 1, n), jnp.float32),
        ],
    )(x, loct, scale)


def _scatter_sc(xt, idx3, hs):
    b, c1, n = xt.shape
    pairs = b * c1
    iters = (pairs + _NW - 1) // _NW
    groups = n // _L
    sh = hs.bit_length() - 1
    mesh = plsc.VectorSubcoreMesh(core_axis_name="c", subcore_axis_name="s")

    @functools.partial(
        pl.kernel,
        mesh=mesh,
        compiler_params=pltpu.CompilerParams(needs_layout_passes=False),
        out_type=jax.ShapeDtypeStruct((b, c1, hs, hs), jnp.float32),
        scratch_types=[
            pltpu.VMEM((n,), jnp.int32),
            pltpu.VMEM((n,), jnp.float32),
            pltpu.VMEM((hs, hs), jnp.float32),
            pltpu.SemaphoreType.DMA,
            pltpu.SemaphoreType.DMA,
        ],
    )
    def scatter_kernel(xt_hbm, idx_hbm, out_hbm, idx_v, val_v, acc_v,
                       sem_i, sem_v):
        wid = lax.axis_index("s") * _NC + lax.axis_index("c")

        def pair_body(k, _):
            p = k * _NW + wid

            @pl.when(p < pairs)
            def _():
                bb = lax.div(p, c1)
                cc = lax.rem(p, c1)
                cp_i = pltpu.async_copy(idx_hbm.at[bb, 0], idx_v, sem_i)
                cp_v = pltpu.async_copy(xt_hbm.at[bb, cc], val_v, sem_v)

                zero = jnp.zeros((_L,), jnp.float32)

                def zero_body(z, _):
                    for u in range(hs // _L):
                        acc_v[z, pl.ds(u * _L, _L)] = zero
                    return ()

                lax.fori_loop(0, hs, zero_body, ())
                cp_i.wait()
                cp_v.wait()

                def scat_body(g, _):
                    base = g * (4 * _L)
                    for u in range(4):
                        o = base + u * _L
                        iv = idx_v[pl.ds(o, _L)]
                        vv = val_v[pl.ds(o, _L)]
                        plsc.addupdate_scatter(
                            acc_v,
                            [lax.shift_right_logical(iv, sh),
                             lax.bitwise_and(iv, hs - 1)],
                            vv)
                    return ()

                lax.fori_loop(0, groups // 4, scat_body, ())
                pltpu.sync_copy(acc_v, out_hbm.at[bb, cc])

            return ()

        lax.fori_loop(0, iters, pair_body, ())

    return scatter_kernel(xt, idx3)


def _sep3(t, w0, w1):
    p = jnp.pad(t, ((0, 0), (1, 1), (0, 0)))
    t = w0 * p[:, :-2, :] + w1 * p[:, 1:-1, :] + w0 * p[:, 2:, :]
    p = jnp.pad(t, ((0, 0), (0, 0), (1, 1)))
    return w0 * p[:, :, :-2] + w1 * p[:, :, 1:-1] + w0 * p[:, :, 2:]


def _finish_body(f_ref, m_ref, o_ref, *, w0, w1):
    cnt = m_ref[0, 0]
    mask = (cnt > 0).astype(jnp.float32)
    feat = f_ref[0] / (cnt + 1e-6)[None] * mask[None]
    cf = _sep3(feat, w0, w1)
    cm = _sep3(mask[None], w0, w1)
    interp = cf / (cm + 1e-6) * (cm > 0).astype(jnp.float32)
    o_ref[0] = feat + (1.0 - mask)[None] * interp


def _finish(accum4):
    b, c1, hs, ws = accum4.shape
    c = c1 - 1
    cch = 12
    a = math.exp(-1.0 / (2.0 * _SIGMA * _SIGMA))
    w1 = 1.0 / (2.0 * a + 1.0)
    w0 = a * w1
    return pl.pallas_call(
        functools.partial(_finish_body, w0=w0, w1=w1),
        grid=(b, c // cch),
        in_specs=[
            pl.BlockSpec((1, cch, hs, ws), lambda i, j: (i, j, 0, 0)),
            pl.BlockSpec((1, 1, hs, ws), lambda i, j: (i, c, 0, 0)),
        ],
        out_specs=pl.BlockSpec((1, cch, hs, ws), lambda i, j: (i, j, 0, 0)),
        out_shape=jax.ShapeDtypeStruct((b, c, hs, ws), jnp.float32),
    )(accum4, accum4)


def kernel(x, loc, h, w):
    b, n, c = x.shape
    hs = math.isqrt(n) * _SCALE_FACTOR
    scale = jnp.stack([jnp.asarray(w, jnp.float32) * _SCALE_FACTOR,
                       jnp.asarray(h, jnp.float32) * _SCALE_FACTOR]
                      ).reshape(2, 1)
    idx3, xt = _prep(x, jnp.swapaxes(loc, 1, 2), scale)
    accum = _scatter_sc(xt, idx3, hs)
    return _finish(accum)


# speedup vs baseline: 2.7838x; 1.0606x over previous
import functools
import math

import jax
import jax.numpy as jnp
from jax import lax
from jax.experimental import pallas as pl
from jax.experimental.pallas import tpu as pltpu
from jax.experimental.pallas import tpu_sc as plsc

_SCALE_FACTOR = 2
_SIGMA = 2.0

_NC = 2
_NS = 16
_NW = _NC * _NS
_L = 16


def _prep_body(x_ref, loct_ref, scale_ref, idx_ref, xt_ref):
    n, c = x_ref.shape[1], x_ref.shape[2]
    xb = x_ref[0]
    eye = (lax.broadcasted_iota(jnp.int32, (c, c), 0)
           == lax.broadcasted_iota(jnp.int32, (c, c), 1)).astype(jnp.float32)
    xt = lax.dot_general(eye, xb, (((1,), (1,)), ((), ())),
                         precision=lax.Precision.HIGHEST,
                         preferred_element_type=jnp.float32)
    xt_ref[0] = jnp.concatenate(
        [xt, jnp.ones((1, n), jnp.float32)], axis=0)

    l = jnp.clip(loct_ref[0], -1.0, 1.0)
    s = scale_ref[...]
    pix = 0.5 * (l + 1.0) * s - 0.5
    ri = jnp.round(pix).astype(jnp.int32)
    ri = jnp.clip(ri, 0, s.astype(jnp.int32) - 1)
    wv = jnp.concatenate(
        [jnp.ones((1, 1), jnp.int32), s.astype(jnp.int32)[0:1, :]], axis=0)
    idx_ref[0, 0] = jnp.sum(ri * wv, axis=0)


def _prep(x, loct, scale):
    b, n, c = x.shape
    return pl.pallas_call(
        _prep_body,
        grid=(b,),
        in_specs=[
            pl.BlockSpec((1, n, c), lambda i: (i, 0, 0)),
            pl.BlockSpec((1, 2, n), lambda i: (i, 0, 0)),
            pl.BlockSpec((2, 1), lambda i: (0, 0)),
        ],
        out_specs=[
            pl.BlockSpec((1, 1, n), lambda i: (i, 0, 0)),
            pl.BlockSpec((1, c + 1, n), lambda i: (i, 0, 0)),
        ],
        out_shape=[
            jax.ShapeDtypeStruct((b, 1, n), jnp.int32),
            jax.ShapeDtypeStruct((b, c + 1, n), jnp.float32),
        ],
    )(x, loct, scale)


def _scatter_sc(xt, idx3, hs):
    b, c1, n = xt.shape
    pairs = b * c1
    iters = (pairs + _NW - 1) // _NW
    groups = n // _L
    sh = hs.bit_length() - 1
    mesh = plsc.VectorSubcoreMesh(core_axis_name="c", subcore_axis_name="s")

    @functools.partial(
        pl.kernel,
        mesh=mesh,
        compiler_params=pltpu.CompilerParams(needs_layout_passes=False),
        out_type=jax.ShapeDtypeStruct((b, c1, hs, hs), jnp.float32),
        scratch_types=[
            pltpu.VMEM((n,), jnp.int32),
            pltpu.VMEM((n,), jnp.float32),
            pltpu.VMEM((hs, hs), jnp.float32),
            pltpu.SemaphoreType.DMA,
            pltpu.SemaphoreType.DMA,
        ],
    )
    def scatter_kernel(xt_hbm, idx_hbm, out_hbm, idx_v, val_v, acc_v,
                       sem_i, sem_v):
        wid = lax.axis_index("s") * _NC + lax.axis_index("c")

        def pair_body(k, _):
            p = k * _NW + wid

            @pl.when(p < pairs)
            def _():
                bb = lax.div(p, c1)
                cc = lax.rem(p, c1)
                cp_i = pltpu.async_copy(idx_hbm.at[bb, 0], idx_v, sem_i)
                cp_v = pltpu.async_copy(xt_hbm.at[bb, cc], val_v, sem_v)

                zero = jnp.zeros((_L,), jnp.float32)

                def zero_body(z, _):
                    for u in range(hs // _L):
                        acc_v[z, pl.ds(u * _L, _L)] = zero
                    return ()

                lax.fori_loop(0, hs, zero_body, ())
                cp_i.wait()
                cp_v.wait()

                def scat_body(g, _):
                    base = g * (4 * _L)
                    for u in range(4):
                        o = base + u * _L
                        iv = idx_v[pl.ds(o, _L)]
                        vv = val_v[pl.ds(o, _L)]
                        plsc.addupdate_scatter(
                            acc_v,
                            [lax.shift_right_logical(iv, sh),
                             lax.bitwise_and(iv, hs - 1)],
                            vv)
                    return ()

                lax.fori_loop(0, groups // 4, scat_body, ())
                pltpu.sync_copy(acc_v, out_hbm.at[bb, cc])

            return ()

        lax.fori_loop(0, iters, pair_body, ())

    return scatter_kernel(xt, idx3)


def _sep3(t, w0, w1, mw):
    p = jnp.pad(t, ((0, 0), (1, 1), (0, 0)))
    t = w0 * p[:, :-2, :] + w1 * p[:, 1:-1, :] + w0 * p[:, 2:, :]
    return lax.dot_general(t, mw, (((2,), (0,)), ((), ())),
                           precision=lax.Precision.HIGHEST,
                           preferred_element_type=jnp.float32)


def _finish_body(f_ref, m_ref, o_ref, *, w0, w1):
    ws = m_ref.shape[3]
    d = (lax.broadcasted_iota(jnp.int32, (ws, ws), 0)
         - lax.broadcasted_iota(jnp.int32, (ws, ws), 1))
    mw = (w1 * (d == 0).astype(jnp.float32)
          + w0 * (jnp.abs(d) == 1).astype(jnp.float32))
    cnt = m_ref[0, 0]
    mask = (cnt > 0).astype(jnp.float32)
    rcp = mask / (cnt + 1e-6)
    feat = f_ref[0] * rcp[None]
    cf = _sep3(feat, w0, w1, mw)
    cm = _sep3(mask[None], w0, w1, mw)
    mrcp = (cm > 0).astype(jnp.float32) / (cm + 1e-6)
    o_ref[0] = feat + (1.0 - mask)[None] * (cf * mrcp)


def _finish(accum4):
    b, c1, hs, ws = accum4.shape
    c = c1 - 1
    cch = 12
    a = math.exp(-1.0 / (2.0 * _SIGMA * _SIGMA))
    w1 = 1.0 / (2.0 * a + 1.0)
    w0 = a * w1
    return pl.pallas_call(
        functools.partial(_finish_body, w0=w0, w1=w1),
        grid=(b, c // cch),
        in_specs=[
            pl.BlockSpec((1, cch, hs, ws), lambda i, j: (i, j, 0, 0)),
            pl.BlockSpec((1, 1, hs, ws), lambda i, j: (i, c, 0, 0)),
        ],
        out_specs=pl.BlockSpec((1, cch, hs, ws), lambda i, j: (i, j, 0, 0)),
        out_shape=jax.ShapeDtypeStruct((b, c, hs, ws), jnp.float32),
    )(accum4, accum4)


def kernel(x, loc, h, w):
    b, n, c = x.shape
    hs = math.isqrt(n) * _SCALE_FACTOR
    scale = jnp.stack([jnp.asarray(w, jnp.float32) * _SCALE_FACTOR,
                       jnp.asarray(h, jnp.float32) * _SCALE_FACTOR]
                      ).reshape(2, 1)
    idx3, xt = _prep(x, jnp.swapaxes(loc, 1, 2), scale)
    accum = _scatter_sc(xt, idx3, hs)
    return _finish(accum)


# speedup vs baseline: 2.8680x; 1.0302x over previous
import functools
import math

import jax
import jax.numpy as jnp
from jax import lax
from jax.experimental import pallas as pl
from jax.experimental.pallas import tpu as pltpu
from jax.experimental.pallas import tpu_sc as plsc

_SCALE_FACTOR = 2
_SIGMA = 2.0

_NC = 2
_NS = 16
_NW = _NC * _NS
_L = 16


def _prep_body(x_ref, loct_ref, scale_ref, idx_ref, xt_ref):
    n, c = x_ref.shape[1], x_ref.shape[2]
    xb = x_ref[0]
    eye = (lax.broadcasted_iota(jnp.int32, (c, c), 0)
           == lax.broadcasted_iota(jnp.int32, (c, c), 1)).astype(jnp.float32)
    xt = lax.dot_general(eye, xb, (((1,), (1,)), ((), ())),
                         precision=lax.Precision.HIGHEST,
                         preferred_element_type=jnp.float32)
    xt_ref[0] = jnp.concatenate(
        [xt, jnp.ones((1, n), jnp.float32)], axis=0)

    l = jnp.clip(loct_ref[0], -1.0, 1.0)
    s = scale_ref[...]
    pix = 0.5 * (l + 1.0) * s - 0.5
    ri = jnp.round(pix).astype(jnp.int32)
    ri = jnp.clip(ri, 0, s.astype(jnp.int32) - 1)
    wv = jnp.concatenate(
        [jnp.ones((1, 1), jnp.int32), s.astype(jnp.int32)[0:1, :]], axis=0)
    idx_ref[0, 0] = jnp.sum(ri * wv, axis=0)


def _prep(x, loct, scale):
    b, n, c = x.shape
    return pl.pallas_call(
        _prep_body,
        grid=(b,),
        in_specs=[
            pl.BlockSpec((1, n, c), lambda i: (i, 0, 0)),
            pl.BlockSpec((1, 2, n), lambda i: (i, 0, 0)),
            pl.BlockSpec((2, 1), lambda i: (0, 0)),
        ],
        out_specs=[
            pl.BlockSpec((1, 1, n), lambda i: (i, 0, 0)),
            pl.BlockSpec((1, c + 1, n), lambda i: (i, 0, 0)),
        ],
        out_shape=[
            jax.ShapeDtypeStruct((b, 1, n), jnp.int32),
            jax.ShapeDtypeStruct((b, c + 1, n), jnp.float32),
        ],
    )(x, loct, scale)


def _scatter_sc(xt, idx3, hs):
    b, c1, n = xt.shape
    pairs = b * c1
    iters = (pairs + _NW - 1) // _NW
    groups = n // _L
    sh = hs.bit_length() - 1
    mesh = plsc.VectorSubcoreMesh(core_axis_name="c", subcore_axis_name="s")

    @functools.partial(
        pl.kernel,
        mesh=mesh,
        compiler_params=pltpu.CompilerParams(needs_layout_passes=False),
        out_type=jax.ShapeDtypeStruct((b, c1, hs, hs), jnp.float32),
        scratch_types=[
            pltpu.VMEM((n,), jnp.int32),
            pltpu.VMEM((n,), jnp.float32),
            pltpu.VMEM((hs, hs), jnp.float32),
            pltpu.SemaphoreType.DMA,
            pltpu.SemaphoreType.DMA,
            pltpu.SemaphoreType.DMA,
            pltpu.SemaphoreType.DMA,
            pltpu.SemaphoreType.DMA,
            pltpu.SemaphoreType.DMA,
        ],
    )
    def scatter_kernel(xt_hbm, idx_hbm, out_hbm, idx_v, val_v, acc_v,
                       sem_i, sem_v, so0, so1, so2, so3):
        wid = lax.axis_index("s") * _NC + lax.axis_index("c")
        sems_o = (so0, so1, so2, so3)
        nch = len(sems_o)
        rows = hs // nch
        zero = jnp.zeros((_L,), jnp.float32)

        def zero_rows(r0, nr):
            def zb(z, _):
                for u in range(hs // _L):
                    acc_v[z, pl.ds(u * _L, _L)] = zero
                return ()
            lax.fori_loop(r0, r0 + nr, zb, ())

        def issue_in(p):
            bb = lax.div(p, c1)
            cc = lax.rem(p, c1)
            pltpu.async_copy(idx_hbm.at[bb, 0], idx_v, sem_i)
            pltpu.async_copy(xt_hbm.at[bb, cc], val_v, sem_v)

        issue_in(wid)
        zero_rows(0, hs)

        def pair_body(k, _):
            p = k * _NW + wid

            @pl.when(p < pairs)
            def _():
                bb = lax.div(p, c1)
                cc = lax.rem(p, c1)
                pltpu.make_async_copy(idx_hbm.at[bb, 0], idx_v, sem_i).wait()
                pltpu.make_async_copy(xt_hbm.at[bb, cc], val_v, sem_v).wait()

                def scat_body(g, _):
                    base = g * (4 * _L)
                    for u in range(4):
                        o = base + u * _L
                        iv = idx_v[pl.ds(o, _L)]
                        vv = val_v[pl.ds(o, _L)]
                        plsc.addupdate_scatter(
                            acc_v,
                            [lax.shift_right_logical(iv, sh),
                             lax.bitwise_and(iv, hs - 1)],
                            vv)
                    return ()

                lax.fori_loop(0, groups // 4, scat_body, ())

                @pl.when(p + _NW < pairs)
                def _():
                    issue_in(p + _NW)

                for i in range(nch):
                    pltpu.async_copy(
                        acc_v.at[pl.ds(i * rows, rows)],
                        out_hbm.at[bb, cc, pl.ds(i * rows, rows)],
                        sems_o[i])
                for i in range(nch):
                    pltpu.make_async_copy(
                        acc_v.at[pl.ds(i * rows, rows)],
                        out_hbm.at[bb, cc, pl.ds(i * rows, rows)],
                        sems_o[i]).wait()
                    zero_rows(i * rows, rows)

            return ()

        lax.fori_loop(0, iters, pair_body, ())

    return scatter_kernel(xt, idx3)


def _sep3(t, w0, w1, mw):
    p = jnp.pad(t, ((0, 0), (1, 1), (0, 0)))
    t = w0 * p[:, :-2, :] + w1 * p[:, 1:-1, :] + w0 * p[:, 2:, :]
    return lax.dot_general(t, mw, (((2,), (0,)), ((), ())),
                           precision=lax.Precision.HIGHEST,
                           preferred_element_type=jnp.float32)


def _finish_body(f_ref, m_ref, o_ref, *, w0, w1):
    ws = m_ref.shape[3]
    d = (lax.broadcasted_iota(jnp.int32, (ws, ws), 0)
         - lax.broadcasted_iota(jnp.int32, (ws, ws), 1))
    mw = (w1 * (d == 0).astype(jnp.float32)
          + w0 * (jnp.abs(d) == 1).astype(jnp.float32))
    cnt = m_ref[0, 0]
    mask = (cnt > 0).astype(jnp.float32)
    rcp = mask / (cnt + 1e-6)
    feat = f_ref[0] * rcp[None]
    cf = _sep3(feat, w0, w1, mw)
    cm = _sep3(mask[None], w0, w1, mw)
    mrcp = (cm > 0).astype(jnp.float32) / (cm + 1e-6)
    o_ref[0] = feat + (1.0 - mask)[None] * (cf * mrcp)


def _finish(accum4):
    b, c1, hs, ws = accum4.shape
    c = c1 - 1
    cch = 12
    a = math.exp(-1.0 / (2.0 * _SIGMA * _SIGMA))
    w1 = 1.0 / (2.0 * a + 1.0)
    w0 = a * w1
    return pl.pallas_call(
        functools.partial(_finish_body, w0=w0, w1=w1),
        grid=(b, c // cch),
        in_specs=[
            pl.BlockSpec((1, cch, hs, ws), lambda i, j: (i, j, 0, 0)),
            pl.BlockSpec((1, 1, hs, ws), lambda i, j: (i, c, 0, 0)),
        ],
        out_specs=pl.BlockSpec((1, cch, hs, ws), lambda i, j: (i, j, 0, 0)),
        out_shape=jax.ShapeDtypeStruct((b, c, hs, ws), jnp.float32),
    )(accum4, accum4)


def kernel(x, loc, h, w):
    b, n, c = x.shape
    hs = math.isqrt(n) * _SCALE_FACTOR
    scale = jnp.stack([jnp.asarray(w, jnp.float32) * _SCALE_FACTOR,
                       jnp.asarray(h, jnp.float32) * _SCALE_FACTOR]
                      ).reshape(2, 1)
    idx3, xt = _prep(x, jnp.swapaxes(loc, 1, 2), scale)
    accum = _scatter_sc(xt, idx3, hs)
    return _finish(accum)
